# baseline (speedup 1.0000x reference)
import functools

import jax
import jax.numpy as jnp
from jax import lax
from jax.experimental import pallas as pl
from jax.experimental.pallas import tpu as pltpu
from jax.experimental.pallas import tpu_sc as plsc

_NC = 2
_NS = 16
_NW = _NC * _NS

_B = 4096 * 16
_D = 768
_BPW = _B // _NW
_C = 64
_NCHUNK = _BPW // _C


@functools.partial(
    pl.kernel,
    out_type=jax.ShapeDtypeStruct((_B, _D), jnp.float32),
    mesh=plsc.VectorSubcoreMesh(core_axis_name="c", subcore_axis_name="s"),
    scratch_types=[
        pltpu.VMEM((_BPW,), jnp.int32),
        pltpu.VMEM((_C, _D), jnp.float32),
        pltpu.VMEM((_C, _D), jnp.float32),
        pltpu.SemaphoreType.DMA,
        pltpu.SemaphoreType.DMA,
    ],
)
def _gather_rows(idx_hbm, table_hbm, out_hbm, idx_v, buf0, buf1, sem0, sem1):
    wid = lax.axis_index("s") * _NC + lax.axis_index("c")
    base = wid * _BPW
    pltpu.sync_copy(idx_hbm.at[pl.ds(base, _BPW)], idx_v)

    bufs = (buf0, buf1)
    sems = (sem0, sem1)

    def _start(c, slot):
        pltpu.async_copy(
            table_hbm.at[idx_v.at[pl.ds(c * _C, _C)]], bufs[slot], sems[slot]
        )

    def _wait(c, slot):
        pltpu.make_async_copy(
            table_hbm.at[idx_v.at[pl.ds(c * _C, _C)]], bufs[slot], sems[slot]
        ).wait()

    _start(0, 0)

    def body(i, _):
        for slot in range(2):
            c = 2 * i + slot

            @pl.when(c + 1 < _NCHUNK)
            def _():
                _start(c + 1, 1 - slot)

            _wait(c, slot)
            pltpu.sync_copy(bufs[slot], out_hbm.at[pl.ds(base + c * _C, _C)])
        return ()

    lax.fori_loop(0, _NCHUNK // 2, body, ())


def kernel(x, table):
    idx = x.reshape(-1).astype(jnp.int32)
    out = _gather_rows(idx, table)
    return out.reshape(x.shape + (table.shape[1],))


# speedup vs baseline: 7.3656x; 7.3656x over previous
import functools

import jax
import jax.numpy as jnp
from jax import lax
from jax.experimental import pallas as pl
from jax.experimental.pallas import tpu as pltpu
from jax.experimental.pallas import tpu_sc as plsc

_NC = 2
_NS = 16
_NW = _NC * _NS

_B = 4096 * 16
_D = 768
_BPW = _B // _NW
_G = _BPW // 16


@functools.partial(
    pl.kernel,
    out_type=jax.ShapeDtypeStruct((_B, _D), jnp.float32),
    mesh=plsc.VectorSubcoreMesh(core_axis_name="c", subcore_axis_name="s"),
    scratch_types=[
        pltpu.VMEM((_BPW,), jnp.int32),
        pltpu.VMEM((16, _D), jnp.float32),
        pltpu.SemaphoreType.DMA,
    ],
)
def _gather_rows(idx_hbm, table_hbm, out_hbm, idx_v, table_v, sem):
    wid = lax.axis_index("s") * _NC + lax.axis_index("c")
    base = wid * _BPW
    pltpu.sync_copy(table_hbm, table_v)
    pltpu.sync_copy(idx_hbm.at[pl.ds(base, _BPW)], idx_v)

    def group(g, _):
        vidx = idx_v[pl.ds(g * 16, 16)]
        for r in range(16):
            i = vidx[r]
            pltpu.async_copy(table_v.at[i], out_hbm.at[base + g * 16 + r], sem)
        return ()

    lax.fori_loop(0, _G, group, ())

    def drain(g, _):
        for _r in range(16):
            pltpu.make_async_copy(table_v.at[0], out_hbm.at[base], sem).wait()
        return ()

    lax.fori_loop(0, _G, drain, ())


def kernel(x, table):
    idx = x.reshape(-1).astype(jnp.int32)
    out = _gather_rows(idx, table)
    return out.reshape(x.shape + (table.shape[1],))


# speedup vs baseline: 7.8497x; 1.0657x over previous
import functools

import jax
import jax.numpy as jnp
from jax import lax
from jax.experimental import pallas as pl
from jax.experimental.pallas import tpu as pltpu
from jax.experimental.pallas import tpu_sc as plsc

_NC = 2
_NS = 16
_NW = _NC * _NS

_B = 4096 * 16
_D = 768
_BPW = _B // _NW
_G = _BPW // 16


@functools.partial(
    pl.kernel,
    out_type=jax.ShapeDtypeStruct((_B, _D), jnp.float32),
    mesh=plsc.VectorSubcoreMesh(core_axis_name="c", subcore_axis_name="s"),
    scratch_types=[
        pltpu.VMEM((_BPW,), jnp.int32),
        pltpu.VMEM((16, _D), jnp.float32),
        pltpu.SemaphoreType.DMA,
    ],
)
def _gather_rows(idx_hbm, table_hbm, out_hbm, idx_v, table_v, sem):
    wid = lax.axis_index("s") * _NC + lax.axis_index("c")
    base = wid * _BPW
    pltpu.sync_copy(table_hbm, table_v)
    pltpu.sync_copy(idx_hbm.at[pl.ds(base, _BPW)], idx_v)

    def group(g, _):
        vidx = idx_v[pl.ds(g * 16, 16)]
        for r in range(16):
            i = vidx[r]
            pltpu.async_copy(table_v.at[i], out_hbm.at[base + g * 16 + r], sem)
        return ()

    lax.fori_loop(0, _G, group, ())

    def drain(g, _):
        pltpu.make_async_copy(
            table_v, out_hbm.at[pl.ds(base + g * 16, 16)], sem
        ).wait()
        return ()

    lax.fori_loop(0, _G, drain, ())


def kernel(x, table):
    idx = x.reshape(-1).astype(jnp.int32)
    out = _gather_rows(idx, table)
    return out.reshape(x.shape + (table.shape[1],))
